# baseline (speedup 1.0000x reference)
import functools

import jax
import jax.numpy as jnp
from jax import lax
from jax.experimental import pallas as pl
from jax.experimental.pallas import tpu as pltpu
from jax.experimental.pallas import tpu_sc as plsc

B = 16384
D = 16
IDX_CHUNK = 128


def _gather_body(b_per_w, n_chunks,
                 uidx_hbm, iidx_hbm, utab_hbm, itab_hbm,
                 uout_hbm, iout_hbm,
                 uidx_v, iidx_v, urows_v, irows_v, sem):
    wid = lax.axis_index("s") * 2 + lax.axis_index("c")
    row_base = wid * n_chunks
    base = wid * b_per_w

    pltpu.sync_copy(uidx_hbm.at[pl.ds(row_base, n_chunks)], uidx_v)
    pltpu.sync_copy(iidx_hbm.at[pl.ds(row_base, n_chunks)], iidx_v)

    copies = []
    for j in range(n_chunks):
        copies.append(pltpu.async_copy(
            utab_hbm.at[uidx_v.at[j]],
            urows_v.at[pl.ds(j * IDX_CHUNK, IDX_CHUNK)], sem))
        copies.append(pltpu.async_copy(
            itab_hbm.at[iidx_v.at[j]],
            irows_v.at[pl.ds(j * IDX_CHUNK, IDX_CHUNK)], sem))
    for c in copies:
        c.wait()

    pltpu.sync_copy(urows_v, uout_hbm.at[pl.ds(base, b_per_w)])
    pltpu.sync_copy(irows_v, iout_hbm.at[pl.ds(base, b_per_w)])


def _sc_gather(uidx2d, iidx2d, utab, itab):
    info = plsc.get_sparse_core_info()
    nw = info.num_cores * info.num_subcores
    b_per_w = B // nw
    n_chunks = b_per_w // IDX_CHUNK
    mesh = plsc.VectorSubcoreMesh(core_axis_name="c", subcore_axis_name="s")
    f = pl.kernel(
        functools.partial(_gather_body, b_per_w, n_chunks),
        mesh=mesh,
        compiler_params=pltpu.CompilerParams(use_tc_tiling_on_sc=False),
        out_type=[
            jax.ShapeDtypeStruct((B, D), jnp.float32),
            jax.ShapeDtypeStruct((B, D), jnp.float32),
        ],
        scratch_types=[
            pltpu.VMEM((n_chunks, IDX_CHUNK), jnp.int32),
            pltpu.VMEM((n_chunks, IDX_CHUNK), jnp.int32),
            pltpu.VMEM((b_per_w, D), jnp.float32),
            pltpu.VMEM((b_per_w, D), jnp.float32),
            pltpu.SemaphoreType.DMA,
        ],
    )
    return f(uidx2d, iidx2d, utab, itab)


def _mlp_body(u_ref, i_ref, w1u_ref, w1i_ref, b1_ref, w2t_ref, b2_ref,
              w3_ref, b3_ref, out_ref):
    x = (jnp.dot(u_ref[...], w1u_ref[...], preferred_element_type=jnp.float32)
         + jnp.dot(i_ref[...], w1i_ref[...], preferred_element_type=jnp.float32)
         + b1_ref[...])
    h1 = jnp.maximum(x, 0.0)
    h2 = jnp.maximum(
        jnp.dot(h1, w2t_ref[...], preferred_element_type=jnp.float32)
        + b2_ref[...], 0.0)
    out_ref[...] = jnp.sum(h2 * w3_ref[...], axis=1, keepdims=True) + b3_ref[...]


def _tc_mlp(u_emb, i_emb, w1u_t, w1i_t, b1, w2t, b2, w3, b3):
    blk = 2048
    grid = (B // blk,)
    full = lambda g: (0, 0)
    return pl.pallas_call(
        _mlp_body,
        grid=grid,
        in_specs=[
            pl.BlockSpec((blk, D), lambda g: (g, 0)),
            pl.BlockSpec((blk, D), lambda g: (g, 0)),
            pl.BlockSpec((D, 64), full),
            pl.BlockSpec((D, 64), full),
            pl.BlockSpec((1, 64), full),
            pl.BlockSpec((64, 32), full),
            pl.BlockSpec((1, 32), full),
            pl.BlockSpec((1, 32), full),
            pl.BlockSpec((1, 1), full),
        ],
        out_specs=pl.BlockSpec((blk, 1), lambda g: (g, 0)),
        out_shape=jax.ShapeDtypeStruct((B, 1), jnp.float32),
    )(u_emb, i_emb, w1u_t, w1i_t, b1, w2t, b2, w3, b3)


def kernel(user, item, user_table, item_table, W1, b1, W2, b2, W3, b3):
    uidx2d = user.astype(jnp.int32).reshape(B // IDX_CHUNK, IDX_CHUNK)
    iidx2d = item.astype(jnp.int32).reshape(B // IDX_CHUNK, IDX_CHUNK)
    u_emb, i_emb = _sc_gather(uidx2d, iidx2d, user_table, item_table)
    w1u_t = W1[:, :D].T
    w1i_t = W1[:, D:].T
    return _tc_mlp(u_emb, i_emb, w1u_t, w1i_t, b1.reshape(1, 64),
                   W2.T, b2.reshape(1, 32), W3, b3.reshape(1, 1))
